# speedup vs baseline: 1.4288x; 1.4288x over previous
import functools

import jax
import jax.numpy as jnp
from jax import lax
from jax.experimental import pallas as pl
from jax.experimental.pallas import tpu as pltpu
from jax.experimental.pallas import tpu_sc as plsc

NUM_CORES = 2
NUM_SUBCORES = 16
NUM_WORKERS = NUM_CORES * NUM_SUBCORES
LANES = 16


def _make_kernel(n_rows, d, chunk):
    assert n_rows % (NUM_WORKERS * chunk) == 0
    rows_per_worker = n_rows // NUM_WORKERS
    n_chunks = rows_per_worker // chunk
    d_vregs = d // LANES

    mesh = plsc.VectorSubcoreMesh(
        core_axis_name="c", subcore_axis_name="s")

    @functools.partial(
        pl.kernel,
        mesh=mesh,
        out_type=jax.ShapeDtypeStruct((n_rows, d), jnp.float32),
        scratch_types=[
            pltpu.VMEM((chunk,), jnp.int32),
            pltpu.VMEM((chunk,), jnp.int32),
            pltpu.VMEM((chunk, d), jnp.float32),
            pltpu.VMEM((chunk, d), jnp.float32),
            pltpu.SemaphoreType.DMA,
            pltpu.SemaphoreType.DMA,
        ],
    )
    def kern(tok_ids_hbm, pos_ids_hbm, tok_tab_hbm, pos_tab_hbm, out_hbm,
             tid_v, pid_v, tok_buf, pos_buf, sem_t, sem_p):
        wid = lax.axis_index("s") * NUM_CORES + lax.axis_index("c")
        base = wid * rows_per_worker

        def chunk_body(j, carry):
            start = base + j * chunk
            pltpu.sync_copy(tok_ids_hbm.at[pl.ds(start, chunk)], tid_v)
            pltpu.sync_copy(pos_ids_hbm.at[pl.ds(start, chunk)], pid_v)
            cp_t = pltpu.async_copy(tok_tab_hbm.at[tid_v], tok_buf, sem_t)
            cp_p = pltpu.async_copy(pos_tab_hbm.at[pid_v], pos_buf, sem_p)
            cp_t.wait()
            cp_p.wait()

            def row_body(r, carry2):
                for c in range(d_vregs):
                    sl = pl.ds(c * LANES, LANES)
                    plsc.addupdate(tok_buf.at[r, sl], pos_buf[r, sl])
                return carry2

            lax.fori_loop(0, chunk, row_body, 0)
            pltpu.sync_copy(tok_buf, out_hbm.at[pl.ds(start, chunk)])
            return carry

        lax.fori_loop(0, n_chunks, chunk_body, 0)

    return kern


def kernel(input_ids, position_ids, token_table, position_table):
    b, w = input_ids.shape
    v, d = token_table.shape
    n_rows = b * w
    flat_tok = input_ids.reshape(n_rows).astype(jnp.int32)
    flat_pos = position_ids.reshape(n_rows).astype(jnp.int32)
    kern = _make_kernel(n_rows, d, chunk=56)
    out = kern(flat_tok, flat_pos, token_table, position_table)
    return out.reshape(b, w, d)
